# baseline (speedup 1.0000x reference)
import functools

import jax
import jax.numpy as jnp
from jax import lax
from jax.experimental import pallas as pl
from jax.experimental.pallas import tpu as pltpu
from jax.experimental.pallas import tpu_sc as plsc

NUM_CORES = 2
NUM_SUBCORES = 16
NW = NUM_CORES * NUM_SUBCORES

RCHUNK = 4
NBUF = 4
NB2 = 2 * NBUF


def _gather_kernel(rows_per_w, keys_hbm, table_hbm, out_hbm, idx_v, rows_v,
                   gsem, wsem):
    n_slabs = rows_per_w // RCHUNK
    wid = lax.axis_index("s") * NUM_CORES + lax.axis_index("c")
    base = wid * rows_per_w
    pltpu.sync_copy(keys_hbm.at[pl.ds(base, rows_per_w)], idx_v)

    def start_gather(c, b):
        for r in range(RCHUNK):
            pltpu.async_copy(
                table_hbm.at[idx_v.at[c * RCHUNK + r]],
                rows_v.at[b, r], gsem.at[b])

    def wait_gather(c, b):
        for r in range(RCHUNK):
            pltpu.make_async_copy(
                table_hbm.at[idx_v.at[c * RCHUNK + r]],
                rows_v.at[b, r], gsem.at[b]).wait()

    def start_write(c, b):
        pltpu.async_copy(
            rows_v.at[b], out_hbm.at[pl.ds(base + c * RCHUNK, RCHUNK)],
            wsem.at[b])

    def wait_write(c, b):
        pltpu.make_async_copy(
            rows_v.at[b], out_hbm.at[pl.ds(base + c * RCHUNK, RCHUNK)],
            wsem.at[b]).wait()

    for b in range(NBUF):
        start_gather(b, b)

    for c in range(NBUF):
        wait_gather(c, c)
        start_write(c, c)
        start_gather(c + NBUF, c + NBUF)

    @pl.loop(NBUF, n_slabs - NBUF, step=NB2)
    def _(i):
        for k in range(NB2):
            c = i + k
            b = (NBUF + k) % NB2
            bn = (b + NBUF) % NB2
            wait_gather(c, b)
            start_write(c, b)
            wait_write(c - NBUF, bn)
            start_gather(c + NBUF, bn)

    for k in range(NBUF):
        c = n_slabs - NBUF + k
        b = c % NB2
        wait_gather(c, b)
        start_write(c, b)

    for j in range(NB2):
        c = n_slabs - NB2 + j
        wait_write(c, c % NB2)


def kernel(keys, table):
    b, f = keys.shape
    v, d = table.shape
    rows_per_w = b // NW

    mesh = plsc.VectorSubcoreMesh(core_axis_name="c", subcore_axis_name="s")
    out = pl.kernel(
        functools.partial(_gather_kernel, rows_per_w),
        out_type=jax.ShapeDtypeStruct((b, f, d), table.dtype),
        mesh=mesh,
        scratch_types=[
            pltpu.VMEM((rows_per_w, f), jnp.int32),
            pltpu.VMEM((NB2, RCHUNK, f, d), jnp.float32),
            pltpu.SemaphoreType.DMA((NB2,)),
            pltpu.SemaphoreType.DMA((NB2,)),
        ],
        compiler_params=pltpu.CompilerParams(use_tc_tiling_on_sc=False),
    )(keys, table)
    return out
